# Initial kernel scaffold; baseline (speedup 1.0000x reference)
import jax
import jax.numpy as jnp
from jax.experimental import pallas as pl


def kernel(input, weight):
    raise NotImplementedError("write your pallas kernel here")


# speedup vs baseline: 12.1323x; 12.1323x over previous
import functools

import jax
import jax.numpy as jnp
from jax import lax
from jax.experimental import pallas as pl
from jax.experimental.pallas import tpu as pltpu
from jax.experimental.pallas import tpu_sc as plsc

B = 16384
S = 20
D = 32
NC, NS, L = 2, 16, 16
NW = NC * NS
VPW = B * S // NW
FPW = VPW * D
ZCHUNK = 20480
NZ = FPW // ZCHUNK
NVEC = VPW // L
GROUP = 256
NGROUP = VPW // GROUP
WROW = 128
WPACK = WROW // D

C0 = 1.0
C1 = 0.8535533905932737
C2 = 0.5
C3 = 0.14644660940672624


@functools.partial(
    pl.kernel,
    out_type=jax.ShapeDtypeStruct((B * S * D,), jnp.float32),
    mesh=plsc.VectorSubcoreMesh(core_axis_name="c", subcore_axis_name="s"),
    scratch_types=[
        pltpu.VMEM((VPW + L,), jnp.int32),
        pltpu.VMEM((ZCHUNK,), jnp.float32),
        pltpu.VMEM((2 * L,), jnp.int32),
        pltpu.VMEM((L, WROW), jnp.float32),
        pltpu.VMEM((D,), jnp.float32),
        pltpu.SemaphoreType.DMA,
        pltpu.SemaphoreType.DMA,
    ],
)
def _sc_embed(inp_hbm, w_hbm, out_hbm,
              inp_v, zero_v, tree_v, rows_v, row_v, zsem, gsem):
    wid = lax.axis_index("s") * NC + lax.axis_index("c")
    base = wid * VPW
    out_base = wid * FPW

    pltpu.sync_copy(inp_hbm.at[pl.ds(base, VPW)], inp_v.at[pl.ds(0, VPW)])
    inp_v[pl.ds(VPW, L)] = jnp.zeros((L,), jnp.int32)

    zf = jnp.zeros((L,), jnp.float32)

    def _zinit(i, carry):
        zero_v[pl.ds(i * L, L)] = zf
        return carry

    lax.fori_loop(0, ZCHUNK // L, _zinit, 0)

    zcopies = []
    for k in range(NZ):
        zcopies.append(
            pltpu.async_copy(
                zero_v, out_hbm.at[pl.ds(out_base + k * ZCHUNK, ZCHUNK)], zsem
            )
        )

    def _tree_min(m):
        for sh in (8, 4, 2, 1):
            tree_v[pl.ds(0, L)] = m
            m = jnp.minimum(m, tree_v[pl.ds(sh, L)])
        return m[0]

    UNROLL = 8

    def _minscan(i, m):
        for u in range(UNROLL):
            m = jnp.minimum(m, inp_v[pl.ds((i * UNROLL + u) * L, L)])
        return m

    m0 = inp_v[pl.ds(0, L)]
    gmin = _tree_min(lax.fori_loop(1, NVEC // UNROLL, _minscan, m0))

    for c in zcopies:
        c.wait()

    def _patch(pos, v):
        b_loc = pos // S
        p0 = jnp.clip(v - 3, 0, 13)
        idx = inp_v[pl.ds(b_loc * S + p0, L)]
        pltpu.async_copy(w_hbm.at[idx >> 2], rows_v, gsem).wait()
        acc_lo = jnp.zeros((L,), jnp.float32)
        acc_hi = jnp.zeros((L,), jnp.float32)
        for r7 in range(7):
            j = jnp.abs(v - (p0 + r7))
            c = jnp.where(
                j == 0, jnp.float32(C0),
                jnp.where(j == 1, jnp.float32(C1),
                          jnp.where(j == 2, jnp.float32(C2),
                                    jnp.where(j == 3, jnp.float32(C3),
                                              jnp.float32(0.0)))))
            cb = jnp.full((L,), c, jnp.float32)
            q32 = (idx[r7] & (WPACK - 1)) * D
            acc_lo = acc_lo + cb * rows_v[r7, pl.ds(q32, L)]
            acc_hi = acc_hi + cb * rows_v[r7, pl.ds(q32 + L, L)]
        row_v[pl.ds(0, L)] = acc_lo
        row_v[pl.ds(L, L)] = acc_hi
        pltpu.sync_copy(row_v, out_hbm.at[pl.ds((base + pos) * D, D)])

    @pl.when(gmin <= 22)
    def _rescan():
        def _group(gi, carry):
            gm = inp_v[pl.ds(gi * GROUP, L)]
            for t in range(1, GROUP // L):
                gm = jnp.minimum(gm, inp_v[pl.ds(gi * GROUP + t * L, L)])
            gmin_s = _tree_min(gm)

            @pl.when(gmin_s <= 22)
            def _():
                def _pos(r, c2):
                    pos = gi * GROUP + r
                    v = inp_v[pl.ds(pos, L)][0]

                    @pl.when(v <= 22)
                    def _():
                        _patch(pos, v)

                    return c2

                lax.fori_loop(0, GROUP, _pos, 0)

            return carry

        lax.fori_loop(0, NGROUP, _group, 0)


def kernel(input, weight):
    out_flat = _sc_embed(input.reshape(-1), weight.reshape(-1, WROW))
    return out_flat.reshape(B, S, D)


# speedup vs baseline: 37.6204x; 3.1008x over previous
import functools

import jax
import jax.numpy as jnp
from jax import lax
from jax.experimental import pallas as pl
from jax.experimental.pallas import tpu as pltpu
from jax.experimental.pallas import tpu_sc as plsc

B = 16384
S = 20
D = 32
NC, NS, L = 2, 16, 16
NW = NC * NS
BPW = B // NW
VPW = BPW * S
BBLK = 128
NBLK = BPW // BBLK
SH = S // 2
CHW = SH * D * BBLK
NVEC = VPW // L
WCOL = 128

C0 = 1.0
C1 = 0.8535533905932737
C2 = 0.5
C3 = 0.14644660940672624


@functools.partial(
    pl.kernel,
    out_type=jax.ShapeDtypeStruct((S, D, B), jnp.float32),
    mesh=plsc.VectorSubcoreMesh(core_axis_name="c", subcore_axis_name="s"),
    scratch_types=[
        pltpu.VMEM((VPW + L,), jnp.int32),
        pltpu.VMEM((SH, D, BBLK), jnp.float32),
        pltpu.VMEM((SH, D, BBLK), jnp.float32),
        pltpu.VMEM((D + 1, WCOL), jnp.float32),
        pltpu.VMEM((2 * L,), jnp.int32),
        pltpu.SemaphoreType.DMA,
        pltpu.SemaphoreType.DMA,
    ],
)
def _sc_embed(inp_hbm, wt_hbm, out_hbm,
              inp_v, zbuf, mbuf, wblk, tree_v, zsem, gsem):
    wid = lax.axis_index("s") * NC + lax.axis_index("c")
    base_b = wid * BPW

    stage = pltpu.async_copy(
        inp_hbm.at[pl.ds(wid * VPW, VPW)], inp_v.at[pl.ds(0, VPW)], gsem)

    zf = jnp.zeros((L,), jnp.float32)

    def _zero_chunk(buf):
        def body(i, carry):
            si = i // (D * BBLK // L)
            r = i % (D * BBLK // L)
            d = r // (BBLK // L)
            k = r % (BBLK // L)
            buf[si, d, pl.ds(k * L, L)] = zf
            return carry

        lax.fori_loop(0, SH * D * BBLK // L, body, 0)

    _zero_chunk(zbuf)
    stage.wait()
    inp_v[pl.ds(VPW, L)] = jnp.zeros((L,), jnp.int32)

    def _tree_min(m):
        for sh in (8, 4, 2, 1):
            tree_v[pl.ds(0, L)] = m
            m = jnp.minimum(m, tree_v[pl.ds(sh, L)])
        return m[0]

    def _minscan(i, m):
        for u in range(8):
            m = jnp.minimum(m, inp_v[pl.ds((i * 8 + u) * L, L)])
        return m

    gmin = _tree_min(lax.fori_loop(1, NVEC // 8, _minscan, inp_v[pl.ds(0, L)]))

    bb0 = pl.multiple_of(base_b, BBLK)

    @pl.when(gmin > 22)
    def _all_clean():
        copies = []
        for c in range(NBLK):
            for h in range(2):
                copies.append(pltpu.async_copy(
                    zbuf,
                    out_hbm.at[pl.ds(h * SH, SH), :, pl.ds(bb0 + c * BBLK, BBLK)],
                    zsem))
        for cp in copies:
            cp.wait()

    @pl.when(gmin <= 22)
    def _per_block():
        def _patch(bflat, si, j, v):
            p0 = jnp.clip(v - 3, 0, 13)
            acc_lo = jnp.zeros((L,), jnp.float32)
            acc_hi = jnp.zeros((L,), jnp.float32)
            for r7 in range(7):
                widx = inp_v[pl.ds(bflat * S + p0 + r7, L)][0]
                qa = pl.multiple_of((widx >> 7) << 7, WCOL)
                q = widx - qa
                pltpu.async_copy(wt_hbm.at[:, pl.ds(qa, WCOL)],
                                 wblk.at[pl.ds(0, D), :], gsem).wait()
                jj = jnp.abs(v - (p0 + r7))
                cf = jnp.where(
                    jj == 0, jnp.float32(C0),
                    jnp.where(jj == 1, jnp.float32(C1),
                              jnp.where(jj == 2, jnp.float32(C2),
                                        jnp.where(jj == 3, jnp.float32(C3),
                                                  jnp.float32(0.0)))))
                iota = lax.iota(jnp.int32, L)
                xlo = jnp.zeros((L,), jnp.float32)
                xhi = jnp.zeros((L,), jnp.float32)
                for d in range(L):
                    xlo = jnp.where(iota == d,
                                    jnp.full((L,), wblk[d, pl.ds(q, L)][0],
                                             jnp.float32), xlo)
                for d in range(L):
                    xhi = jnp.where(iota == d,
                                    jnp.full((L,), wblk[L + d, pl.ds(q, L)][0],
                                             jnp.float32), xhi)
                cb = jnp.full((L,), cf, jnp.float32)
                acc_lo = acc_lo + cb * xlo
                acc_hi = acc_hi + cb * xhi
            iota = lax.iota(jnp.int32, L)
            lane = j & (L - 1)
            b_al = (j >> 4) << 4
            for d in range(L):
                cur = mbuf[si, d, pl.ds(b_al, L)]
                val = jnp.full((L,), acc_lo[d], jnp.float32)
                mbuf[si, d, pl.ds(b_al, L)] = jnp.where(iota == lane, val, cur)
            for d in range(L):
                cur = mbuf[si, L + d, pl.ds(b_al, L)]
                val = jnp.full((L,), acc_hi[d], jnp.float32)
                mbuf[si, L + d, pl.ds(b_al, L)] = jnp.where(iota == lane, val, cur)

        def _block(c, carry):
            def _bscan(i, m):
                for u in range(8):
                    m = jnp.minimum(
                        m, inp_v[pl.ds(c * BBLK * S + (i * 8 + u) * L, L)])
                return m

            m0 = inp_v[pl.ds(c * BBLK * S, L)]
            bmin = _tree_min(lax.fori_loop(1, BBLK * S // L // 8, _bscan, m0))

            cb_off = pl.multiple_of(bb0 + c * BBLK, BBLK)

            @pl.when(bmin > 22)
            def _clean():
                for h in range(2):
                    pltpu.async_copy(
                        zbuf,
                        out_hbm.at[pl.ds(h * SH, SH), :, pl.ds(cb_off, BBLK)],
                        zsem).wait()

            @pl.when(bmin <= 22)
            def _dirty():
                def _half(h, carry2):
                    _zero_chunk(mbuf)

                    def _pos(t, carry3):
                        j = t // SH
                        si = t % SH
                        bflat = c * BBLK + j
                        v = inp_v[pl.ds(bflat * S + h * SH + si, L)][0]

                        @pl.when(v <= 22)
                        def _():
                            _patch(bflat, si, j, v)

                        return carry3

                    lax.fori_loop(0, BBLK * SH, _pos, 0)
                    pltpu.async_copy(
                        mbuf,
                        out_hbm.at[pl.ds(h * SH, SH), :, pl.ds(cb_off, BBLK)],
                        zsem).wait()
                    return carry2

                lax.fori_loop(0, 2, _half, 0)

            return carry

        lax.fori_loop(0, NBLK, _block, 0)


def kernel(input, weight):
    out_t = _sc_embed(input.reshape(-1), weight.T)
    return out_t.transpose(2, 0, 1)


# speedup vs baseline: 112.0288x; 2.9779x over previous
import functools

import jax
import jax.numpy as jnp
from jax import lax
from jax.experimental import pallas as pl
from jax.experimental.pallas import tpu as pltpu
from jax.experimental.pallas import tpu_sc as plsc

B = 16384
S = 20
D = 32
NC, NS, L = 2, 16, 16
SH = S // NC
DPS = D // NS
BPW = B // NS
VPW = BPW * S
NVEC = VPW // L
WCOL = 128
OBLK = 128

C0 = 1.0
C1 = 0.8535533905932737
C2 = 0.5
C3 = 0.14644660940672624


@functools.partial(
    pl.kernel,
    out_type=jax.ShapeDtypeStruct((S, D, B), jnp.float32),
    mesh=plsc.VectorSubcoreMesh(core_axis_name="c", subcore_axis_name="s"),
    scratch_types=[
        pltpu.VMEM((VPW + L,), jnp.int32),
        pltpu.VMEM((D, BPW), jnp.float32),
        pltpu.VMEM((D + 1, WCOL), jnp.float32),
        pltpu.VMEM((D, OBLK), jnp.float32),
        pltpu.VMEM((2 * L,), jnp.int32),
        pltpu.SemaphoreType.DMA,
        pltpu.SemaphoreType.DMA,
    ],
)
def _sc_embed(inp_hbm, wt_hbm, out_hbm,
              inp_v, zslab, wblk, pbuf, tree_v, zsem, gsem):
    core = lax.axis_index("c")
    sub = lax.axis_index("s")

    stage = pltpu.async_copy(
        inp_hbm.at[pl.ds(sub * VPW, VPW)], inp_v.at[pl.ds(0, VPW)], gsem)

    zf = jnp.zeros((L,), jnp.float32)

    def _zinit(i, carry):
        for u in range(8):
            t = i * 8 + u
            zslab[t // (BPW // L), pl.ds((t % (BPW // L)) * L, L)] = zf
        return carry

    lax.fori_loop(0, D * BPW // L // 8, _zinit, 0)

    s0 = core * SH
    b0 = pl.multiple_of(sub * BPW, OBLK)
    fills = []
    for t in range(SH):
        fills.append(pltpu.async_copy(
            zslab, out_hbm.at[s0 + t, :, pl.ds(b0, BPW)], zsem))

    stage.wait()
    inp_v[pl.ds(VPW, L)] = jnp.zeros((L,), jnp.int32)

    def _tree_min(m):
        for sh in (8, 4, 2, 1):
            tree_v[pl.ds(0, L)] = m
            m = jnp.minimum(m, tree_v[pl.ds(sh, L)])
        return m[0]

    def _minscan(i, m):
        for u in range(8):
            m = jnp.minimum(m, inp_v[pl.ds((i * 8 + u) * L, L)])
        return m

    gmin = _tree_min(lax.fori_loop(1, NVEC // 8, _minscan, inp_v[pl.ds(0, L)]))

    for f in fills:
        f.wait()

    @pl.when(gmin <= 22)
    def _patch_scan():
        def _patch(b_loc, s, v):
            p0 = jnp.clip(v - 3, 0, 13)
            acc_lo = jnp.zeros((L,), jnp.float32)
            acc_hi = jnp.zeros((L,), jnp.float32)
            for r7 in range(7):
                widx = inp_v[pl.ds(b_loc * S + p0 + r7, L)][0]
                qa = pl.multiple_of((widx >> 7) << 7, WCOL)
                q = widx - qa
                pltpu.async_copy(wt_hbm.at[:, pl.ds(qa, WCOL)],
                                 wblk.at[pl.ds(0, D), :], gsem).wait()
                jj = jnp.abs(v - (p0 + r7))
                cf = jnp.where(
                    jj == 0, jnp.float32(C0),
                    jnp.where(jj == 1, jnp.float32(C1),
                              jnp.where(jj == 2, jnp.float32(C2),
                                        jnp.where(jj == 3, jnp.float32(C3),
                                                  jnp.float32(0.0)))))
                iota = lax.iota(jnp.int32, L)
                xlo = jnp.zeros((L,), jnp.float32)
                xhi = jnp.zeros((L,), jnp.float32)
                for d in range(L):
                    xlo = jnp.where(iota == d,
                                    jnp.full((L,), wblk[d, pl.ds(q, L)][0],
                                             jnp.float32), xlo)
                for d in range(L):
                    xhi = jnp.where(iota == d,
                                    jnp.full((L,), wblk[L + d, pl.ds(q, L)][0],
                                             jnp.float32), xhi)
                cb = jnp.full((L,), cf, jnp.float32)
                acc_lo = acc_lo + cb * xlo
                acc_hi = acc_hi + cb * xhi
            b = sub * BPW + b_loc
            b_al = pl.multiple_of((b >> 7) << 7, OBLK)
            jb = b - b_al
            pltpu.async_copy(out_hbm.at[s, :, pl.ds(b_al, OBLK)],
                             pbuf, gsem).wait()
            iota = lax.iota(jnp.int32, L)
            lane = jb & (L - 1)
            c_al = (jb >> 4) << 4
            for d in range(L):
                cur = pbuf[d, pl.ds(c_al, L)]
                val = jnp.full((L,), acc_lo[d], jnp.float32)
                pbuf[d, pl.ds(c_al, L)] = jnp.where(iota == lane, val, cur)
            for d in range(L):
                cur = pbuf[L + d, pl.ds(c_al, L)]
                val = jnp.full((L,), acc_hi[d], jnp.float32)
                pbuf[L + d, pl.ds(c_al, L)] = jnp.where(iota == lane, val, cur)
            pltpu.async_copy(pbuf, out_hbm.at[s, :, pl.ds(b_al, OBLK)],
                             gsem).wait()

        def _group(g, carry):
            def _gscan(i, m):
                for u in range(4):
                    m = jnp.minimum(
                        m, inp_v[pl.ds(g * 256 + (i * 4 + u) * L, L)])
                return m

            gm = _tree_min(
                lax.fori_loop(1, 4, _gscan, inp_v[pl.ds(g * 256, L)]))

            @pl.when(gm <= 22)
            def _():
                def _pos(t, c2):
                    v = inp_v[pl.ds(t, L)][0]
                    b_loc = t // S
                    s = t - b_loc * S

                    @pl.when((v <= 22) & (s >= core * SH)
                             & (s < core * SH + SH))
                    def _():
                        _patch(b_loc, s, v)

                    return c2

                lax.fori_loop(g * 256, (g + 1) * 256, _pos, 0)

            return carry

        lax.fori_loop(0, VPW // 256, _group, 0)


def kernel(input, weight):
    out_t = _sc_embed(input.reshape(-1), weight.T)
    return out_t.transpose(2, 0, 1)
